# baseline (speedup 1.0000x reference)
import functools

import jax
import jax.numpy as jnp
from jax import lax
from jax.experimental import pallas as pl
from jax.experimental.pallas import tpu as pltpu
from jax.experimental.pallas import tpu_sc as plsc

NUM_GRAPHS = 1024
N = 10000
D = 512
L = 16
NC = 2
NS = 16
NW = NC * NS
GPW = NUM_GRAPHS // NW
RB = 40
SB = 40
NUM_SB = N // SB
NCHUNK = D // L

_mesh = plsc.VectorSubcoreMesh(
    core_axis_name="c", subcore_axis_name="s", num_cores=NC, num_subcores=NS)


@functools.partial(
    pl.kernel,
    out_type=jax.ShapeDtypeStruct((NUM_GRAPHS, D), jnp.float32),
    mesh=_mesh,
    scratch_types=[
        pltpu.VMEM((N + L,), jnp.int32),
        pltpu.VMEM((GPW, D), jnp.float32),
        pltpu.VMEM((2, RB, D), jnp.float32),
        pltpu.SemaphoreType.DMA,
        pltpu.SemaphoreType.DMA,
    ],
)
def _segsum(x_hbm, idx_hbm, out_hbm, idx_v, acc_v, rows_v, s0, s1):
  wid = lax.axis_index("s") * NC + lax.axis_index("c")
  g_lo = wid * GPW
  g_hi = g_lo + GPW
  pltpu.sync_copy(idx_hbm, idx_v.at[pl.ds(0, N)])
  idx_v[pl.ds(N, L)] = jnp.full((L,), NUM_GRAPHS, jnp.int32)

  def lower_bound(target):
    def bs_body(_, carry):
      lo, hi = carry
      mid = (lo + hi) // 2
      v = idx_v[pl.ds(mid, L)][0]
      pred = (v < target) & (lo < hi)
      return jnp.where(pred, mid + 1, lo), jnp.where(pred, hi, mid)

    lo, _ = lax.fori_loop(0, 14, bs_body, (jnp.int32(0), jnp.int32(N)))
    return lo

  start = lower_bound(g_lo)
  cnt = lower_bound(g_hi) - start

  def zero_body(r, carry):
    for j in range(NCHUNK):
      acc_v[r, pl.ds(j * L, L)] = jnp.zeros((L,), jnp.float32)
    return carry

  lax.fori_loop(0, GPW, zero_body, 0)

  end = start + cnt
  blk0 = start // RB
  nblk = (end + RB - 1) // RB - blk0
  sem = (s0, s1)

  def start_blk(i, slot):
    base = pl.multiple_of((blk0 + i) * RB, 8)
    pltpu.async_copy(x_hbm.at[pl.ds(base, RB)], rows_v.at[slot], sem[slot])

  def wait_blk(slot):
    pltpu.make_async_copy(
        x_hbm.at[pl.ds(0, RB)], rows_v.at[slot], sem[slot]).wait()

  pl.when(nblk > 0)(lambda: start_blk(0, 0))
  pl.when(nblk > 1)(lambda: start_blk(1, 1))

  def outer(g, carry):
    for b in range(2):
      i = g * 2 + b

      @pl.when(i < nblk)
      def _(i=i, b=b):
        wait_blk(b)
        base = (blk0 + i) * RB
        r_lo = jnp.maximum(start - base, 0)
        r_hi = jnp.minimum(end - base, RB)

        def row_body(r, rc):
          lg = idx_v[pl.ds(base + r, L)][0] - g_lo
          for j in range(NCHUNK):
            sl = pl.ds(j * L, L)
            plsc.addupdate(acc_v.at[lg, sl], rows_v[b, r, sl])
          return rc

        lax.fori_loop(r_lo, r_hi, row_body, 0)
        pl.when(i + 2 < nblk)(lambda: start_blk(i + 2, b))

    return carry

  lax.fori_loop(0, (nblk + 1) // 2, outer, 0)
  pltpu.sync_copy(acc_v, out_hbm.at[pl.ds(g_lo, GPW)])


def _mlp_body(vn_ref, w1, b1, w2, b2, w3, b3, w4, b4, out_ref):
  vn = vn_ref[...]
  h = jnp.maximum(
      jnp.dot(vn, w1[...], preferred_element_type=jnp.float32) + b1[...], 0.0)
  h = jnp.maximum(
      jnp.dot(h, w2[...], preferred_element_type=jnp.float32) + b2[...], 0.0)
  h = jnp.maximum(
      jnp.dot(h, w3[...], preferred_element_type=jnp.float32) + b3[...], 0.0)
  out_ref[...] = (
      jnp.dot(h, w4[...], preferred_element_type=jnp.float32) + b4[...])


_mlp = pl.pallas_call(
    _mlp_body,
    out_shape=jax.ShapeDtypeStruct((NUM_GRAPHS, D), jnp.float32),
)


MAXB = 8


@functools.partial(
    pl.kernel,
    out_type=jax.ShapeDtypeStruct((N, D), jnp.float32),
    mesh=_mesh,
    scratch_types=[
        pltpu.VMEM((MAXB * SB,), jnp.int32),
        pltpu.VMEM((2, SB, D), jnp.float32),
        pltpu.VMEM((2, SB, D), jnp.float32),
        pltpu.SemaphoreType.DMA,
        pltpu.SemaphoreType.DMA,
        pltpu.SemaphoreType.DMA,
        pltpu.SemaphoreType.DMA,
        pltpu.SemaphoreType.DMA,
        pltpu.SemaphoreType.DMA,
    ],
)
def _gather_add(x_hbm, idx_hbm, h_hbm, out_hbm, idx_v, hbuf, xbuf,
                g0, g1, xs0, xs1, o0, o1):
  wid = lax.axis_index("s") * NC + lax.axis_index("c")
  b_lo = wid * NUM_SB // NW
  nb = (wid + 1) * NUM_SB // NW - b_lo
  gsem = (g0, g1)
  xsem = (xs0, xs1)
  osem = (o0, o1)

  idx_base = pl.multiple_of(b_lo * SB, 8)
  pltpu.sync_copy(idx_hbm.at[pl.ds(idx_base, MAXB * SB)], idx_v)

  def start(j):
    slot = j % 2
    base = pl.multiple_of((b_lo + j) * SB, 8)
    gd = pltpu.async_copy(
        h_hbm.at[idx_v.at[pl.ds(j * SB, SB)]], hbuf.at[slot], gsem[slot])
    xd = pltpu.async_copy(x_hbm.at[pl.ds(base, SB)], xbuf.at[slot], xsem[slot])
    return gd, xd

  def finish(j, gd, xd):
    slot = j % 2
    base = pl.multiple_of((b_lo + j) * SB, 8)
    gd.wait()
    xd.wait()

    def row(r, rc):
      for k in range(NCHUNK):
        sl = pl.ds(k * L, L)
        plsc.addupdate(xbuf.at[slot, r, sl], hbuf[slot, r, sl])
      return rc

    lax.fori_loop(0, SB, row, 0)
    return pltpu.async_copy(xbuf.at[slot], out_hbm.at[pl.ds(base, SB)],
                            osem[slot])

  def pipe(nb_s):
    def go():
      descs = {0: start(0)}
      odescs = {}
      for j in range(1, nb_s):
        if j >= 2:
          odescs[j - 2].wait()
        descs[j] = start(j)
        odescs[j - 1] = finish(j - 1, *descs[j - 1])
      odescs[nb_s - 1] = finish(nb_s - 1, *descs[nb_s - 1])
      odescs[nb_s - 2].wait()
      odescs[nb_s - 1].wait()
    return go

  pl.when(nb == 7)(pipe(7))
  pl.when(nb == 8)(pipe(8))


def kernel(x, graph_idx, W1, b1, W2, b2, W3, b3, W4, b4):
  idx = graph_idx.astype(jnp.int32)
  vn = _segsum(x, idx)
  h = _mlp(vn, W1, b1.reshape(1, D), W2, b2.reshape(1, D),
           W3, b3.reshape(1, D), W4, b4.reshape(1, D))
  return _gather_add(x, idx, h)


# speedup vs baseline: 1.2814x; 1.2814x over previous
import functools

import jax
import jax.numpy as jnp
from jax import lax
from jax.experimental import pallas as pl
from jax.experimental.pallas import tpu as pltpu
from jax.experimental.pallas import tpu_sc as plsc

NUM_GRAPHS = 1024
N = 10000
D = 512
L = 16
NC = 2
NS = 16
NW = NC * NS
GPW = NUM_GRAPHS // NW
RB = 40
SB = 40
NUM_SB = N // SB
NCHUNK = D // L

_mesh = plsc.VectorSubcoreMesh(
    core_axis_name="c", subcore_axis_name="s", num_cores=NC, num_subcores=NS)


@functools.partial(
    pl.kernel,
    out_type=jax.ShapeDtypeStruct((NUM_GRAPHS, D), jnp.float32),
    mesh=_mesh,
    scratch_types=[
        pltpu.VMEM((N + L,), jnp.int32),
        pltpu.VMEM((GPW, D), jnp.float32),
        pltpu.VMEM((2, RB, D), jnp.float32),
        pltpu.SemaphoreType.DMA,
        pltpu.SemaphoreType.DMA,
    ],
)
def _segsum(x_hbm, idx_hbm, out_hbm, idx_v, acc_v, rows_v, s0, s1):
  wid = lax.axis_index("s") * NC + lax.axis_index("c")
  g_lo = wid * GPW
  g_hi = g_lo + GPW
  pltpu.sync_copy(idx_hbm, idx_v.at[pl.ds(0, N)])
  idx_v[pl.ds(N, L)] = jnp.full((L,), NUM_GRAPHS, jnp.int32)

  def lower_bound(target):
    def bs_body(_, carry):
      lo, hi = carry
      mid = (lo + hi) // 2
      v = idx_v[pl.ds(mid, L)][0]
      pred = (v < target) & (lo < hi)
      return jnp.where(pred, mid + 1, lo), jnp.where(pred, hi, mid)

    lo, _ = lax.fori_loop(0, 14, bs_body, (jnp.int32(0), jnp.int32(N)))
    return lo

  start = lower_bound(g_lo)
  cnt = lower_bound(g_hi) - start

  def zero_body(r, carry):
    for j in range(NCHUNK):
      acc_v[r, pl.ds(j * L, L)] = jnp.zeros((L,), jnp.float32)
    return carry

  lax.fori_loop(0, GPW, zero_body, 0)

  end = start + cnt
  blk0 = start // RB
  nblk = (end + RB - 1) // RB - blk0
  sem = (s0, s1)

  def start_blk(i, slot):
    base = pl.multiple_of((blk0 + i) * RB, 8)
    pltpu.async_copy(x_hbm.at[pl.ds(base, RB)], rows_v.at[slot], sem[slot])

  def wait_blk(slot):
    pltpu.make_async_copy(
        x_hbm.at[pl.ds(0, RB)], rows_v.at[slot], sem[slot]).wait()

  pl.when(nblk > 0)(lambda: start_blk(0, 0))
  pl.when(nblk > 1)(lambda: start_blk(1, 1))

  def outer(g, carry):
    for b in range(2):
      i = g * 2 + b

      @pl.when(i < nblk)
      def _(i=i, b=b):
        wait_blk(b)
        base = (blk0 + i) * RB
        r_lo = jnp.maximum(start - base, 0)
        r_hi = jnp.minimum(end - base, RB)

        def row_body(r, rc):
          lg = idx_v[pl.ds(base + r, L)][0] - g_lo
          for j0 in range(0, NCHUNK, 8):
            ts = [rows_v[b, r, pl.ds((j0 + k) * L, L)] for k in range(8)]
            for k in range(8):
              plsc.addupdate(acc_v.at[lg, pl.ds((j0 + k) * L, L)], ts[k])
          return rc

        lax.fori_loop(r_lo, r_hi, row_body, 0)
        pl.when(i + 2 < nblk)(lambda: start_blk(i + 2, b))

    return carry

  lax.fori_loop(0, (nblk + 1) // 2, outer, 0)
  pltpu.sync_copy(acc_v, out_hbm.at[pl.ds(g_lo, GPW)])


def _mlp_body(vn_ref, w1, b1, w2, b2, w3, b3, w4, b4, out_ref):
  vn = vn_ref[...]
  h = jnp.maximum(
      jnp.dot(vn, w1[...], preferred_element_type=jnp.float32) + b1[...], 0.0)
  h = jnp.maximum(
      jnp.dot(h, w2[...], preferred_element_type=jnp.float32) + b2[...], 0.0)
  h = jnp.maximum(
      jnp.dot(h, w3[...], preferred_element_type=jnp.float32) + b3[...], 0.0)
  out_ref[...] = (
      jnp.dot(h, w4[...], preferred_element_type=jnp.float32) + b4[...])


_mlp = pl.pallas_call(
    _mlp_body,
    out_shape=jax.ShapeDtypeStruct((NUM_GRAPHS, D), jnp.float32),
)


MAXB = 8


@functools.partial(
    pl.kernel,
    out_type=jax.ShapeDtypeStruct((N, D), jnp.float32),
    mesh=_mesh,
    scratch_types=[
        pltpu.VMEM((MAXB * SB,), jnp.int32),
        pltpu.VMEM((2, SB, D), jnp.float32),
        pltpu.VMEM((2, SB, D), jnp.float32),
        pltpu.SemaphoreType.DMA,
        pltpu.SemaphoreType.DMA,
        pltpu.SemaphoreType.DMA,
        pltpu.SemaphoreType.DMA,
        pltpu.SemaphoreType.DMA,
        pltpu.SemaphoreType.DMA,
    ],
)
def _gather_add(x_hbm, idx_hbm, h_hbm, out_hbm, idx_v, hbuf, xbuf,
                g0, g1, xs0, xs1, o0, o1):
  wid = lax.axis_index("s") * NC + lax.axis_index("c")
  b_lo = wid * NUM_SB // NW
  nb = (wid + 1) * NUM_SB // NW - b_lo
  gsem = (g0, g1)
  xsem = (xs0, xs1)
  osem = (o0, o1)

  idx_base = pl.multiple_of(b_lo * SB, 8)
  pltpu.sync_copy(idx_hbm.at[pl.ds(idx_base, MAXB * SB)], idx_v)

  def start(j):
    slot = j % 2
    base = pl.multiple_of((b_lo + j) * SB, 8)
    gd = pltpu.async_copy(
        h_hbm.at[idx_v.at[pl.ds(j * SB, SB)]], hbuf.at[slot], gsem[slot])
    xd = pltpu.async_copy(x_hbm.at[pl.ds(base, SB)], xbuf.at[slot], xsem[slot])
    return gd, xd

  def finish(j, gd, xd):
    slot = j % 2
    base = pl.multiple_of((b_lo + j) * SB, 8)
    gd.wait()
    xd.wait()

    def row(r, rc):
      for k0 in range(0, NCHUNK, 8):
        ts = [hbuf[slot, r, pl.ds((k0 + k) * L, L)] for k in range(8)]
        for k in range(8):
          plsc.addupdate(xbuf.at[slot, r, pl.ds((k0 + k) * L, L)], ts[k])
      return rc

    lax.fori_loop(0, SB, row, 0)
    return pltpu.async_copy(xbuf.at[slot], out_hbm.at[pl.ds(base, SB)],
                            osem[slot])

  def pipe(nb_s):
    def go():
      descs = {0: start(0)}
      odescs = {}
      for j in range(1, nb_s):
        if j >= 2:
          odescs[j - 2].wait()
        descs[j] = start(j)
        odescs[j - 1] = finish(j - 1, *descs[j - 1])
      odescs[nb_s - 1] = finish(nb_s - 1, *descs[nb_s - 1])
      odescs[nb_s - 2].wait()
      odescs[nb_s - 1].wait()
    return go

  pl.when(nb == 7)(pipe(7))
  pl.when(nb == 8)(pipe(8))


def kernel(x, graph_idx, W1, b1, W2, b2, W3, b3, W4, b4):
  idx = graph_idx.astype(jnp.int32)
  vn = _segsum(x, idx)
  h = _mlp(vn, W1, b1.reshape(1, D), W2, b2.reshape(1, D),
           W3, b3.reshape(1, D), W4, b4.reshape(1, D))
  return _gather_add(x, idx, h)


# speedup vs baseline: 1.4011x; 1.0934x over previous
import functools

import jax
import jax.numpy as jnp
from jax import lax
from jax.experimental import pallas as pl
from jax.experimental.pallas import tpu as pltpu
from jax.experimental.pallas import tpu_sc as plsc

NUM_GRAPHS = 1024
N = 10000
D = 512
L = 16
NC = 2
NS = 16
NW = NC * NS
GPW = NUM_GRAPHS // NW
RB = 40
SB = 40
NUM_SB = N // SB
NCHUNK = D // L

_mesh = plsc.VectorSubcoreMesh(
    core_axis_name="c", subcore_axis_name="s", num_cores=NC, num_subcores=NS)


@functools.partial(
    pl.kernel,
    out_type=jax.ShapeDtypeStruct((NUM_GRAPHS, D), jnp.float32),
    mesh=_mesh,
    scratch_types=[
        pltpu.VMEM((N + L,), jnp.int32),
        pltpu.VMEM((GPW, D), jnp.float32),
        pltpu.VMEM((2, RB, D), jnp.float32),
        pltpu.SemaphoreType.DMA,
        pltpu.SemaphoreType.DMA,
    ],
)
def _segsum(x_hbm, idx_hbm, out_hbm, idx_v, acc_v, rows_v, s0, s1):
  wid = lax.axis_index("s") * NC + lax.axis_index("c")
  g_lo = wid * GPW
  g_hi = g_lo + GPW
  pltpu.sync_copy(idx_hbm, idx_v.at[pl.ds(0, N)])
  idx_v[pl.ds(N, L)] = jnp.full((L,), NUM_GRAPHS, jnp.int32)

  def lower_bound(target):
    def bs_body(_, carry):
      lo, hi = carry
      mid = (lo + hi) // 2
      v = idx_v[pl.ds(mid, L)][0]
      pred = (v < target) & (lo < hi)
      return jnp.where(pred, mid + 1, lo), jnp.where(pred, hi, mid)

    lo, _ = lax.fori_loop(0, 14, bs_body, (jnp.int32(0), jnp.int32(N)))
    return lo

  start = lower_bound(g_lo)
  cnt = lower_bound(g_hi) - start

  def zero_body(r, carry):
    for j in range(NCHUNK):
      acc_v[r, pl.ds(j * L, L)] = jnp.zeros((L,), jnp.float32)
    return carry

  lax.fori_loop(0, GPW, zero_body, 0)

  end = start + cnt
  blk0 = start // RB
  nblk = (end + RB - 1) // RB - blk0
  sem = (s0, s1)

  def start_blk(i, slot):
    base = pl.multiple_of((blk0 + i) * RB, 8)
    pltpu.async_copy(x_hbm.at[pl.ds(base, RB)], rows_v.at[slot], sem[slot])

  def wait_blk(slot):
    pltpu.make_async_copy(
        x_hbm.at[pl.ds(0, RB)], rows_v.at[slot], sem[slot]).wait()

  pl.when(nblk > 0)(lambda: start_blk(0, 0))
  pl.when(nblk > 1)(lambda: start_blk(1, 1))

  def outer(g, carry):
    for b in range(2):
      i = g * 2 + b

      @pl.when(i < nblk)
      def _(i=i, b=b):
        wait_blk(b)
        base = (blk0 + i) * RB
        r_lo = jnp.maximum(start - base, 0)
        r_hi = jnp.minimum(end - base, RB)

        def row_body(r, rc):
          lg = idx_v[pl.ds(base + r, L)][0] - g_lo
          for j0 in range(0, NCHUNK, 8):
            ts = [rows_v[b, r, pl.ds((j0 + k) * L, L)] for k in range(8)]
            for k in range(8):
              plsc.addupdate(acc_v.at[lg, pl.ds((j0 + k) * L, L)], ts[k])
          return rc

        lax.fori_loop(r_lo, r_hi, row_body, 0)
        pl.when(i + 2 < nblk)(lambda: start_blk(i + 2, b))

    return carry

  lax.fori_loop(0, (nblk + 1) // 2, outer, 0)
  pltpu.sync_copy(acc_v, out_hbm.at[pl.ds(g_lo, GPW)])


def _mlp_body(vn_ref, w1, b1, w2, b2, w3, b3, w4, b4, out_ref):
  vn = vn_ref[...]
  h = jnp.maximum(
      jnp.dot(vn, w1[...], preferred_element_type=jnp.float32) + b1[...], 0.0)
  h = jnp.maximum(
      jnp.dot(h, w2[...], preferred_element_type=jnp.float32) + b2[...], 0.0)
  h = jnp.maximum(
      jnp.dot(h, w3[...], preferred_element_type=jnp.float32) + b3[...], 0.0)
  out_ref[...] = (
      jnp.dot(h, w4[...], preferred_element_type=jnp.float32) + b4[...])


_mlp = pl.pallas_call(
    _mlp_body,
    out_shape=jax.ShapeDtypeStruct((NUM_GRAPHS, D), jnp.float32),
)


MAXB = 8
HMAX = 64


@functools.partial(
    pl.kernel,
    out_type=jax.ShapeDtypeStruct((N, D), jnp.float32),
    mesh=_mesh,
    scratch_types=[
        pltpu.VMEM((MAXB * SB + L,), jnp.int32),
        pltpu.VMEM((HMAX, D), jnp.float32),
        pltpu.VMEM((2, SB, D), jnp.float32),
        pltpu.VMEM((2, SB, D), jnp.float32),
        pltpu.SemaphoreType.DMA,
        pltpu.SemaphoreType.DMA,
        pltpu.SemaphoreType.DMA,
        pltpu.SemaphoreType.DMA,
        pltpu.SemaphoreType.DMA,
        pltpu.SemaphoreType.DMA,
    ],
)
def _gather_add(x_hbm, idx_hbm, h_hbm, out_hbm, idx_v, hl_v, hbuf, xbuf,
                g0, g1, xs0, xs1, o0, o1):
  wid = lax.axis_index("s") * NC + lax.axis_index("c")
  b_lo = wid * NUM_SB // NW
  nb = (wid + 1) * NUM_SB // NW - b_lo
  gsem = (g0, g1)
  xsem = (xs0, xs1)
  osem = (o0, o1)

  idx_base = pl.multiple_of(b_lo * SB, 8)
  pltpu.sync_copy(idx_hbm.at[pl.ds(idx_base, MAXB * SB)], idx_v.at[pl.ds(0, MAXB * SB)])

  g_first = idx_v[pl.ds(0, L)][0]
  g_last = idx_v[pl.ds(nb * SB - 1, L)][0]
  hbase = jnp.minimum((g_first // 8) * 8, NUM_GRAPHS - HMAX)
  hbase = pl.multiple_of(hbase, 8)
  linear_ok = (g_last - hbase) < HMAX

  def start(j, linear):
    slot = j % 2
    base = pl.multiple_of((b_lo + j) * SB, 8)
    if not linear:
      pltpu.async_copy(
          h_hbm.at[idx_v.at[pl.ds(j * SB, SB)]], hbuf.at[slot], gsem[slot])
    pltpu.async_copy(x_hbm.at[pl.ds(base, SB)], xbuf.at[slot], xsem[slot])

  def wait_start(j, linear):
    slot = j % 2
    if not linear:
      pltpu.make_async_copy(
          x_hbm.at[pl.ds(0, SB)], hbuf.at[slot], gsem[slot]).wait()
    pltpu.make_async_copy(
        x_hbm.at[pl.ds(0, SB)], xbuf.at[slot], xsem[slot]).wait()

  def finish(j, linear):
    slot = j % 2
    base = pl.multiple_of((b_lo + j) * SB, 8)
    wait_start(j, linear)

    if linear:
      def row(r, rc):
        lg = idx_v[pl.ds(j * SB + r, L)][0] - hbase
        for k0 in range(0, NCHUNK, 8):
          ts = [hl_v[lg, pl.ds((k0 + k) * L, L)] for k in range(8)]
          for k in range(8):
            plsc.addupdate(xbuf.at[slot, r, pl.ds((k0 + k) * L, L)], ts[k])
        return rc
    else:
      def row(r, rc):
        for k0 in range(0, NCHUNK, 8):
          ts = [hbuf[slot, r, pl.ds((k0 + k) * L, L)] for k in range(8)]
          for k in range(8):
            plsc.addupdate(xbuf.at[slot, r, pl.ds((k0 + k) * L, L)], ts[k])
        return rc

    lax.fori_loop(0, SB, row, 0)
    return pltpu.async_copy(xbuf.at[slot], out_hbm.at[pl.ds(base, SB)],
                            osem[slot])

  def pipe(linear):
    def go():
      if linear:
        pltpu.sync_copy(h_hbm.at[pl.ds(hbase, HMAX)], hl_v)
      odescs = {}
      start(0, linear)
      start(1, linear)
      odescs[0] = finish(0, linear)
      for j in range(2, MAXB - 1):
        odescs[j - 2].wait()
        start(j, linear)
        odescs[j - 1] = finish(j - 1, linear)

      @pl.when(nb == MAXB)
      def _():
        odescs[MAXB - 3].wait()
        start(MAXB - 1, linear)
        od6 = finish(MAXB - 2, linear)
        od7 = finish(MAXB - 1, linear)
        od6.wait()
        od7.wait()

      @pl.when(nb == MAXB - 1)
      def _():
        od6 = finish(MAXB - 2, linear)
        odescs[MAXB - 3].wait()
        od6.wait()
    return go

  pl.when(linear_ok)(pipe(True))
  pl.when(jnp.logical_not(linear_ok))(pipe(False))


def kernel(x, graph_idx, W1, b1, W2, b2, W3, b3, W4, b4):
  idx = graph_idx.astype(jnp.int32)
  vn = _segsum(x, idx)
  h = _mlp(vn, W1, b1.reshape(1, D), W2, b2.reshape(1, D),
           W3, b3.reshape(1, D), W4, b4.reshape(1, D))
  return _gather_add(x, idx, h)


# speedup vs baseline: 1.4346x; 1.0240x over previous
import functools

import jax
import jax.numpy as jnp
from jax import lax
from jax.experimental import pallas as pl
from jax.experimental.pallas import tpu as pltpu
from jax.experimental.pallas import tpu_sc as plsc

NUM_GRAPHS = 1024
N = 10000
D = 512
L = 16
NC = 2
NS = 16
NW = NC * NS
GPW = NUM_GRAPHS // NW
RB = 80
SB = 40
NUM_SB = N // SB
NCHUNK = D // L

_mesh = plsc.VectorSubcoreMesh(
    core_axis_name="c", subcore_axis_name="s", num_cores=NC, num_subcores=NS)


@functools.partial(
    pl.kernel,
    out_type=jax.ShapeDtypeStruct((NUM_GRAPHS, D), jnp.float32),
    mesh=_mesh,
    scratch_types=[
        pltpu.VMEM((N + L,), jnp.int32),
        pltpu.VMEM((GPW, D), jnp.float32),
        pltpu.VMEM((2, RB, D), jnp.float32),
        pltpu.SemaphoreType.DMA,
        pltpu.SemaphoreType.DMA,
    ],
)
def _segsum(x_hbm, idx_hbm, out_hbm, idx_v, acc_v, rows_v, s0, s1):
  wid = lax.axis_index("s") * NC + lax.axis_index("c")
  g_lo = wid * GPW
  g_hi = g_lo + GPW
  pltpu.sync_copy(idx_hbm, idx_v.at[pl.ds(0, N)])
  idx_v[pl.ds(N, L)] = jnp.full((L,), NUM_GRAPHS, jnp.int32)

  def lower_bound(target):
    def bs_body(_, carry):
      lo, hi = carry
      mid = (lo + hi) // 2
      v = idx_v[pl.ds(mid, L)][0]
      pred = (v < target) & (lo < hi)
      return jnp.where(pred, mid + 1, lo), jnp.where(pred, hi, mid)

    lo, _ = lax.fori_loop(0, 14, bs_body, (jnp.int32(0), jnp.int32(N)))
    return lo

  start = lower_bound(g_lo)
  cnt = lower_bound(g_hi) - start

  def zero_body(r, carry):
    for j in range(NCHUNK):
      acc_v[r, pl.ds(j * L, L)] = jnp.zeros((L,), jnp.float32)
    return carry

  lax.fori_loop(0, GPW, zero_body, 0)

  end = start + cnt
  blk0 = start // RB
  nblk = (end + RB - 1) // RB - blk0
  sem = (s0, s1)

  def start_blk(i, slot):
    base = pl.multiple_of((blk0 + i) * RB, 8)
    pltpu.async_copy(x_hbm.at[pl.ds(base, RB)], rows_v.at[slot], sem[slot])

  def wait_blk(slot):
    pltpu.make_async_copy(
        x_hbm.at[pl.ds(0, RB)], rows_v.at[slot], sem[slot]).wait()

  pl.when(nblk > 0)(lambda: start_blk(0, 0))
  pl.when(nblk > 1)(lambda: start_blk(1, 1))

  def outer(g, carry):
    for b in range(2):
      i = g * 2 + b

      @pl.when(i < nblk)
      def _(i=i, b=b):
        wait_blk(b)
        base = (blk0 + i) * RB
        r_lo = jnp.maximum(start - base, 0)
        r_hi = jnp.minimum(end - base, RB)

        def one_row(r):
          lg = idx_v[pl.ds(base + r, L)][0] - g_lo
          for j0 in range(0, NCHUNK, 8):
            ts = [rows_v[b, r, pl.ds((j0 + k) * L, L)] for k in range(8)]
            for k in range(8):
              plsc.addupdate(acc_v.at[lg, pl.ds((j0 + k) * L, L)], ts[k])

        nrows = r_hi - r_lo

        def row_pair(i, rc):
          one_row(r_lo + i * 2)
          one_row(r_lo + i * 2 + 1)
          return rc

        lax.fori_loop(0, nrows // 2, row_pair, 0)
        pl.when(nrows % 2 == 1)(lambda: one_row(r_hi - 1))
        pl.when(i + 2 < nblk)(lambda: start_blk(i + 2, b))

    return carry

  lax.fori_loop(0, (nblk + 1) // 2, outer, 0)
  pltpu.sync_copy(acc_v, out_hbm.at[pl.ds(g_lo, GPW)])


def _mlp_body(vn_ref, w1, b1, w2, b2, w3, b3, w4, b4, out_ref):
  def dot16(a, w):
    return jnp.dot(a.astype(jnp.bfloat16), w[...].astype(jnp.bfloat16),
                   preferred_element_type=jnp.float32)

  vn = vn_ref[...]
  h = jnp.maximum(dot16(vn, w1) + b1[...], 0.0)
  h = jnp.maximum(dot16(h, w2) + b2[...], 0.0)
  h = jnp.maximum(dot16(h, w3) + b3[...], 0.0)
  out_ref[...] = dot16(h, w4) + b4[...]


_mlp = pl.pallas_call(
    _mlp_body,
    out_shape=jax.ShapeDtypeStruct((NUM_GRAPHS, D), jnp.float32),
)


MAXB = 8
HMAX = 64


@functools.partial(
    pl.kernel,
    out_type=jax.ShapeDtypeStruct((N, D), jnp.float32),
    mesh=_mesh,
    scratch_types=[
        pltpu.VMEM((MAXB * SB + L,), jnp.int32),
        pltpu.VMEM((HMAX, D), jnp.float32),
        pltpu.VMEM((2, SB, D), jnp.float32),
        pltpu.VMEM((2, SB, D), jnp.float32),
        pltpu.SemaphoreType.DMA,
        pltpu.SemaphoreType.DMA,
        pltpu.SemaphoreType.DMA,
        pltpu.SemaphoreType.DMA,
        pltpu.SemaphoreType.DMA,
        pltpu.SemaphoreType.DMA,
    ],
)
def _gather_add(x_hbm, idx_hbm, h_hbm, out_hbm, idx_v, hl_v, hbuf, xbuf,
                g0, g1, xs0, xs1, o0, o1):
  wid = lax.axis_index("s") * NC + lax.axis_index("c")
  b_lo = wid * NUM_SB // NW
  nb = (wid + 1) * NUM_SB // NW - b_lo
  gsem = (g0, g1)
  xsem = (xs0, xs1)
  osem = (o0, o1)

  idx_base = pl.multiple_of(b_lo * SB, 8)
  pltpu.sync_copy(idx_hbm.at[pl.ds(idx_base, MAXB * SB)], idx_v.at[pl.ds(0, MAXB * SB)])

  g_first = idx_v[pl.ds(0, L)][0]
  g_last = idx_v[pl.ds(nb * SB - 1, L)][0]
  hbase = jnp.minimum((g_first // 8) * 8, NUM_GRAPHS - HMAX)
  hbase = pl.multiple_of(hbase, 8)
  linear_ok = (g_last - hbase) < HMAX

  def start(j, linear):
    slot = j % 2
    base = pl.multiple_of((b_lo + j) * SB, 8)
    if not linear:
      pltpu.async_copy(
          h_hbm.at[idx_v.at[pl.ds(j * SB, SB)]], hbuf.at[slot], gsem[slot])
    pltpu.async_copy(x_hbm.at[pl.ds(base, SB)], xbuf.at[slot], xsem[slot])

  def wait_start(j, linear):
    slot = j % 2
    if not linear:
      pltpu.make_async_copy(
          x_hbm.at[pl.ds(0, SB)], hbuf.at[slot], gsem[slot]).wait()
    pltpu.make_async_copy(
        x_hbm.at[pl.ds(0, SB)], xbuf.at[slot], xsem[slot]).wait()

  def finish(j, linear):
    slot = j % 2
    base = pl.multiple_of((b_lo + j) * SB, 8)
    wait_start(j, linear)

    if linear:
      def row(rr, rc):
        r0 = rr * 2
        r1 = r0 + 1
        lg0 = idx_v[pl.ds(j * SB + r0, L)][0] - hbase
        lg1 = idx_v[pl.ds(j * SB + r1, L)][0] - hbase
        for k0 in range(0, NCHUNK, 8):
          t0 = [hl_v[lg0, pl.ds((k0 + k) * L, L)] for k in range(8)]
          t1 = [hl_v[lg1, pl.ds((k0 + k) * L, L)] for k in range(8)]
          for k in range(8):
            plsc.addupdate(xbuf.at[slot, r0, pl.ds((k0 + k) * L, L)], t0[k])
            plsc.addupdate(xbuf.at[slot, r1, pl.ds((k0 + k) * L, L)], t1[k])
        return rc

      lax.fori_loop(0, SB // 2, row, 0)
    else:
      def row(r, rc):
        for k0 in range(0, NCHUNK, 8):
          ts = [hbuf[slot, r, pl.ds((k0 + k) * L, L)] for k in range(8)]
          for k in range(8):
            plsc.addupdate(xbuf.at[slot, r, pl.ds((k0 + k) * L, L)], ts[k])
        return rc

      lax.fori_loop(0, SB, row, 0)
    return pltpu.async_copy(xbuf.at[slot], out_hbm.at[pl.ds(base, SB)],
                            osem[slot])

  def pipe(linear):
    def go():
      if linear:
        pltpu.sync_copy(h_hbm.at[pl.ds(hbase, HMAX)], hl_v)
      odescs = {}
      start(0, linear)
      start(1, linear)
      odescs[0] = finish(0, linear)
      for j in range(2, MAXB - 1):
        odescs[j - 2].wait()
        start(j, linear)
        odescs[j - 1] = finish(j - 1, linear)

      @pl.when(nb == MAXB)
      def _():
        odescs[MAXB - 3].wait()
        start(MAXB - 1, linear)
        od6 = finish(MAXB - 2, linear)
        od7 = finish(MAXB - 1, linear)
        od6.wait()
        od7.wait()

      @pl.when(nb == MAXB - 1)
      def _():
        od6 = finish(MAXB - 2, linear)
        odescs[MAXB - 3].wait()
        od6.wait()
    return go

  pl.when(linear_ok)(pipe(True))
  pl.when(jnp.logical_not(linear_ok))(pipe(False))


def kernel(x, graph_idx, W1, b1, W2, b2, W3, b3, W4, b4):
  idx = graph_idx.astype(jnp.int32)
  vn = _segsum(x, idx)
  h = _mlp(vn, W1, b1.reshape(1, D), W2, b2.reshape(1, D),
           W3, b3.reshape(1, D), W4, b4.reshape(1, D))
  return _gather_add(x, idx, h)
